# baseline (speedup 1.0000x reference)
import functools

import jax
import jax.numpy as jnp
from jax import lax
from jax.experimental import pallas as pl
from jax.experimental.pallas import tpu as pltpu
from jax.experimental.pallas import tpu_sc as plsc

NC = 2
NS = 16
NW = NC * NS
LANES = 16

CHUNK = 128
GROUP = 8
RING = 2
DCHUNK = 80
ZROWS = 128


def _mesh():
    return plsc.VectorSubcoreMesh(
        core_axis_name="c", subcore_axis_name="s", num_cores=NC, num_subcores=NS
    )


def _make_deg_kernel(E, NPAD):
    e_per = E // NW
    n_chunks = e_per // DCHUNK
    rows_per_tile = NPAD // NS

    @functools.partial(
        pl.kernel,
        out_type=jax.ShapeDtypeStruct((NC, NPAD), jnp.float32),
        mesh=_mesh(),
        scratch_types=[
            pltpu.VMEM((DCHUNK,), jnp.int32),
            pltpu.VMEM((DCHUNK,), jnp.float32),
            pltpu.VMEM((rows_per_tile,), jnp.float32),
            pltpu.VMEM_SHARED((NPAD,), jnp.float32),
        ],
    )
    def deg_kernel(dst_hbm, out_hbm, idx_v, ones_v, zero_v, acc_sh):
        cid = lax.axis_index("c")
        sid = lax.axis_index("s")
        wid = sid * NC + cid

        def fill_ones(i, carry):
            ones_v[pl.ds(i * LANES, LANES)] = jnp.full((LANES,), 1.0, jnp.float32)
            return carry

        lax.fori_loop(0, DCHUNK // LANES, fill_ones, 0)

        def fill_zero(i, carry):
            zero_v[pl.ds(i * LANES, LANES)] = jnp.zeros((LANES,), jnp.float32)
            return carry

        lax.fori_loop(0, rows_per_tile // LANES, fill_zero, 0)

        r0 = sid * rows_per_tile
        pltpu.sync_copy(zero_v, acc_sh.at[pl.ds(r0, rows_per_tile)])
        plsc.subcore_barrier()

        def body(j, carry):
            base = wid * e_per + j * DCHUNK
            pltpu.sync_copy(dst_hbm.at[pl.ds(base, DCHUNK)], idx_v)
            pltpu.sync_copy(ones_v, acc_sh.at[idx_v], add=True)
            return carry

        lax.fori_loop(0, n_chunks, body, 0)

        plsc.subcore_barrier()
        pltpu.sync_copy(
            acc_sh.at[pl.ds(r0, rows_per_tile)],
            out_hbm.at[cid, pl.ds(r0, rows_per_tile)],
        )

    return deg_kernel


def _make_agg_kernel(D, E_pad, NPAD):
    e_per = E_pad // NW
    n_chunks = e_per // CHUNK
    n_groups = n_chunks // GROUP
    rows_per_tile = NPAD // NS

    @functools.partial(
        pl.kernel,
        out_type=jax.ShapeDtypeStruct((NC, NPAD, D), jnp.float32),
        mesh=_mesh(),
        scratch_types=[
            pltpu.VMEM((2, GROUP, CHUNK), jnp.int32),
            pltpu.VMEM((2, GROUP, CHUNK), jnp.int32),
            pltpu.VMEM((RING, CHUNK, D), jnp.float32),
            pltpu.VMEM_SHARED((NPAD, D), jnp.float32),
            pltpu.SemaphoreType.DMA,
            pltpu.SemaphoreType.DMA,
        ]
        + [pltpu.SemaphoreType.DMA] * RING,
    )
    def agg_kernel(g_hbm, src_hbm, dst_hbm, out_hbm,
                   src_v, dst_v, rows_v, acc_sh, is0, is1, *rsems):
        isems = (is0, is1)
        cid = lax.axis_index("c")
        sid = lax.axis_index("s")
        wid = sid * NC + cid

        def idx_start(g, p):
            pltpu.async_copy(src_hbm.at[wid, g], src_v.at[p], isems[p])
            pltpu.async_copy(dst_hbm.at[wid, g], dst_v.at[p], isems[p])

        def idx_wait(g, p):
            pltpu.make_async_copy(src_hbm.at[wid, g], src_v.at[p], isems[p]).wait()
            pltpu.make_async_copy(dst_hbm.at[wid, g], dst_v.at[p], isems[p]).wait()

        def gather_start(p, k, slot):
            pltpu.async_copy(g_hbm.at[src_v.at[p, k]], rows_v.at[slot], rsems[slot])

        def gather_wait(p, k, slot):
            pltpu.make_async_copy(
                g_hbm.at[src_v.at[p, k]], rows_v.at[slot], rsems[slot]
            ).wait()

        def do_group(g, p, start_next_idx, has_next_group):
            for k in range(GROUP):
                slot = k % RING
                gather_wait(p, k, slot)
                pltpu.sync_copy(rows_v.at[slot], acc_sh.at[dst_v.at[p, k]], add=True)
                if k < GROUP - RING:
                    gather_start(p, k + RING, slot)
                else:
                    if has_next_group:
                        if k == GROUP - RING:
                            idx_wait(g + 1, 1 - p)
                        gather_start(1 - p, k - (GROUP - RING), slot)
            if start_next_idx:
                idx_start(g + 2, p)

        idx_start(0, 0)
        idx_start(1, 1)

        zref = rows_v.at[0]

        def fill_zero(i, carry):
            zref[i // (D // LANES), pl.ds((i % (D // LANES)) * LANES, LANES)] = (
                jnp.zeros((LANES,), jnp.float32)
            )
            return carry

        lax.fori_loop(0, CHUNK * (D // LANES), fill_zero, 0)

        r0 = sid * rows_per_tile
        for k in range(rows_per_tile // CHUNK):
            pltpu.sync_copy(zref, acc_sh.at[pl.ds(r0 + k * CHUNK, CHUNK)])

        idx_wait(0, 0)
        for k in range(RING):
            gather_start(0, k, k)
        plsc.subcore_barrier()

        def supergroup(sg, carry):
            do_group(2 * sg, 0, True, True)
            do_group(2 * sg + 1, 1, True, True)
            return carry

        lax.fori_loop(0, (n_groups - 2) // 2, supergroup, 0)
        do_group(n_groups - 2, 0, False, True)
        do_group(n_groups - 1, 1, False, False)

        plsc.subcore_barrier()
        pltpu.sync_copy(
            acc_sh.at[pl.ds(r0, rows_per_tile)],
            out_hbm.at[cid, pl.ds(r0, rows_per_tile)],
        )

    return agg_kernel


def _tc_first(x, W, degT, BN):
    Nn, D = x.shape

    def body(x_ref, w_ref, deg_ref, o_ref):
        deg = deg_ref[:, 0:1] + deg_ref[:, 1:2] + 1.0
        dinv = lax.rsqrt(deg)
        h = jnp.dot(x_ref[...], w_ref[...], preferred_element_type=jnp.float32)
        o_ref[...] = h * dinv

    return pl.pallas_call(
        body,
        grid=(Nn // BN,),
        in_specs=[
            pl.BlockSpec((BN, D), lambda i: (i, 0)),
            pl.BlockSpec((D, D), lambda i: (0, 0)),
            pl.BlockSpec((BN, 2), lambda i: (i, 0)),
        ],
        out_specs=pl.BlockSpec((BN, D), lambda i: (i, 0)),
        out_shape=jax.ShapeDtypeStruct((Nn, D), jnp.float32),
    )(x, W, degT)


def _tc_next(part, g_prev, degT, b, W, BN, final, b_out=None):
    Nn, D = g_prev.shape

    def body(p_ref, g_ref, deg_ref, b_ref, w_ref, bo_ref, o_ref):
        deg = deg_ref[:, 0:1] + deg_ref[:, 1:2] + 1.0
        dinv = lax.rsqrt(deg)
        agg = p_ref[0] + p_ref[1] + g_ref[...]
        h = jnp.maximum(agg * dinv + b_ref[...], 0.0)
        hw = jnp.dot(h, w_ref[...], preferred_element_type=jnp.float32)
        if final:
            o_ref[...] = hw + bo_ref[...]
        else:
            o_ref[...] = hw * dinv

    if b_out is None:
        b_out = jnp.zeros((1, D), jnp.float32)

    return pl.pallas_call(
        body,
        grid=(Nn // BN,),
        in_specs=[
            pl.BlockSpec((2, BN, D), lambda i: (0, i, 0)),
            pl.BlockSpec((BN, D), lambda i: (i, 0)),
            pl.BlockSpec((BN, 2), lambda i: (i, 0)),
            pl.BlockSpec((1, D), lambda i: (0, 0)),
            pl.BlockSpec((D, D), lambda i: (0, 0)),
            pl.BlockSpec((1, D), lambda i: (0, 0)),
        ],
        out_specs=pl.BlockSpec((BN, D), lambda i: (i, 0)),
        out_shape=jax.ShapeDtypeStruct((Nn, D), jnp.float32),
    )(part, g_prev, degT, b, W, b_out)


@jax.jit
def kernel(x, edge_index, W1, b1, W2, b2, W_lin, b_lin):
    Nn, D = x.shape
    E = edge_index.shape[1]
    NPAD = ((Nn + NW * LANES - 1) // (NW * LANES)) * (NW * LANES)
    BN = 1000

    e_tile = ((E // NW) + GROUP * CHUNK - 1) // (GROUP * CHUNK) * (GROUP * CHUNK)
    E_pad = e_tile * NW
    n_groups = e_tile // (GROUP * CHUNK)
    pad = E_pad - E
    src_p = jnp.concatenate([edge_index[0], jnp.zeros((pad,), jnp.int32)])
    dst_p = jnp.concatenate(
        [edge_index[1], jnp.full((pad,), NPAD - 1, jnp.int32)]
    )
    src4 = src_p.reshape(NW, n_groups, GROUP, CHUNK)
    dst4 = dst_p.reshape(NW, n_groups, GROUP, CHUNK)

    deg_parts = _make_deg_kernel(E, NPAD)(edge_index[1])
    degT = deg_parts.T

    g1 = _tc_first(x, W1, degT, BN)
    part1 = _make_agg_kernel(D, E_pad, NPAD)(g1, src4, dst4)

    g2 = _tc_next(part1, g1, degT, b1.reshape(1, D), W2, BN, final=False)
    part2 = _make_agg_kernel(D, E_pad, NPAD)(g2, src4, dst4)

    y = _tc_next(part2, g2, degT, b2.reshape(1, D), W_lin, BN,
                 final=True, b_out=b_lin.reshape(1, D))
    return y


# speedup vs baseline: 2.7822x; 2.7822x over previous
import functools

import jax
import jax.numpy as jnp
from jax import lax
from jax.experimental import pallas as pl
from jax.experimental.pallas import tpu as pltpu
from jax.experimental.pallas import tpu_sc as plsc

NC = 2
NS = 16
NW = NC * NS
LANES = 16

CHUNK = 128
GROUP = 8
RING = 2
DCHUNK = 80
ZROWS = 128


def _mesh():
    return plsc.VectorSubcoreMesh(
        core_axis_name="c", subcore_axis_name="s", num_cores=NC, num_subcores=NS
    )


def _make_deg_kernel(E, NPAD):
    e_per = E // NW
    n_chunks = e_per // DCHUNK
    rows_per_tile = NPAD // NS

    @functools.partial(
        pl.kernel,
        out_type=jax.ShapeDtypeStruct((NC, NPAD), jnp.float32),
        mesh=_mesh(),
        scratch_types=[
            pltpu.VMEM((DCHUNK,), jnp.int32),
            pltpu.VMEM((DCHUNK,), jnp.float32),
            pltpu.VMEM((rows_per_tile,), jnp.float32),
            pltpu.VMEM_SHARED((NPAD,), jnp.float32),
        ],
    )
    def deg_kernel(dst_hbm, out_hbm, idx_v, ones_v, zero_v, acc_sh):
        cid = lax.axis_index("c")
        sid = lax.axis_index("s")
        wid = sid * NC + cid

        def fill_ones(i, carry):
            ones_v[pl.ds(i * LANES, LANES)] = jnp.full((LANES,), 1.0, jnp.float32)
            return carry

        lax.fori_loop(0, DCHUNK // LANES, fill_ones, 0)

        def fill_zero(i, carry):
            zero_v[pl.ds(i * LANES, LANES)] = jnp.zeros((LANES,), jnp.float32)
            return carry

        lax.fori_loop(0, rows_per_tile // LANES, fill_zero, 0)

        r0 = sid * rows_per_tile
        pltpu.sync_copy(zero_v, acc_sh.at[pl.ds(r0, rows_per_tile)])
        plsc.subcore_barrier()

        def body(j, carry):
            base = wid * e_per + j * DCHUNK
            pltpu.sync_copy(dst_hbm.at[pl.ds(base, DCHUNK)], idx_v)
            pltpu.sync_copy(ones_v, acc_sh.at[idx_v], add=True)
            return carry

        lax.fori_loop(0, n_chunks, body, 0)

        plsc.subcore_barrier()
        pltpu.sync_copy(
            acc_sh.at[pl.ds(r0, rows_per_tile)],
            out_hbm.at[cid, pl.ds(r0, rows_per_tile)],
        )

    return deg_kernel


def _make_agg_kernel(D, E_pad, NPAD):
    e_per = E_pad // NW
    n_chunks = e_per // CHUNK
    n_groups = n_chunks // GROUP
    rows_per_tile = NPAD // NS

    @functools.partial(
        pl.kernel,
        out_type=jax.ShapeDtypeStruct((NC, NPAD, D), jnp.float32),
        mesh=_mesh(),
        scratch_types=[
            pltpu.VMEM((2, GROUP, CHUNK), jnp.int32),
            pltpu.VMEM((2, GROUP, CHUNK), jnp.int32),
            pltpu.VMEM((RING, CHUNK, D), jnp.float32),
            pltpu.VMEM_SHARED((NPAD, D), jnp.float32),
            pltpu.SemaphoreType.DMA,
            pltpu.SemaphoreType.DMA,
        ]
        + [pltpu.SemaphoreType.DMA] * RING,
    )
    def agg_kernel(g_hbm, src_hbm, dst_hbm, out_hbm,
                   src_v, dst_v, rows_v, acc_sh, is0, is1, *rsems):
        isems = (is0, is1)
        cid = lax.axis_index("c")
        sid = lax.axis_index("s")
        wid = sid * NC + cid

        def idx_start(g, p):
            pltpu.async_copy(src_hbm.at[wid, g], src_v.at[p], isems[p])
            pltpu.async_copy(dst_hbm.at[wid, g], dst_v.at[p], isems[p])

        def idx_wait(g, p):
            pltpu.make_async_copy(src_hbm.at[wid, g], src_v.at[p], isems[p]).wait()
            pltpu.make_async_copy(dst_hbm.at[wid, g], dst_v.at[p], isems[p]).wait()

        def gather_start(p, k, slot):
            pltpu.async_copy(g_hbm.at[src_v.at[p, k]], rows_v.at[slot], rsems[slot])

        def gather_wait(p, k, slot):
            pltpu.make_async_copy(
                g_hbm.at[src_v.at[p, k]], rows_v.at[slot], rsems[slot]
            ).wait()

        def do_group(g, p, start_next_idx, has_next_group):
            for k in range(GROUP):
                slot = k % RING
                gather_wait(p, k, slot)
                pltpu.sync_copy(rows_v.at[slot], acc_sh.at[dst_v.at[p, k]], add=True)
                if k < GROUP - RING:
                    gather_start(p, k + RING, slot)
                else:
                    if has_next_group:
                        if k == GROUP - RING:
                            idx_wait(g + 1, 1 - p)
                        gather_start(1 - p, k - (GROUP - RING), slot)
            if start_next_idx:
                idx_start(g + 2, p)

        idx_start(0, 0)
        idx_start(1, 1)

        zref = rows_v.at[0]

        def fill_zero(i, carry):
            zref[i // (D // LANES), pl.ds((i % (D // LANES)) * LANES, LANES)] = (
                jnp.zeros((LANES,), jnp.float32)
            )
            return carry

        lax.fori_loop(0, CHUNK * (D // LANES), fill_zero, 0)

        r0 = sid * rows_per_tile
        for k in range(rows_per_tile // CHUNK):
            pltpu.sync_copy(zref, acc_sh.at[pl.ds(r0 + k * CHUNK, CHUNK)])

        idx_wait(0, 0)
        for k in range(RING):
            gather_start(0, k, k)
        plsc.subcore_barrier()

        def supergroup(sg, carry):
            do_group(2 * sg, 0, True, True)
            do_group(2 * sg + 1, 1, True, True)
            return carry

        lax.fori_loop(0, (n_groups - 2) // 2, supergroup, 0)
        do_group(n_groups - 2, 0, False, True)
        do_group(n_groups - 1, 1, False, False)

        plsc.subcore_barrier()
        pltpu.sync_copy(
            acc_sh.at[pl.ds(r0, rows_per_tile)],
            out_hbm.at[cid, pl.ds(r0, rows_per_tile)],
        )

    return agg_kernel


def _tc_first(x, W, degT, BN):
    Nn, D = x.shape

    def body(x_ref, w_ref, deg_ref, o_ref):
        deg = deg_ref[:, 0:1] + deg_ref[:, 1:2] + 1.0
        dinv = lax.rsqrt(deg)
        h = jnp.dot(x_ref[...], w_ref[...], preferred_element_type=jnp.float32)
        o_ref[...] = h * dinv

    return pl.pallas_call(
        body,
        grid=(Nn // BN,),
        in_specs=[
            pl.BlockSpec((BN, D), lambda i: (i, 0)),
            pl.BlockSpec((D, D), lambda i: (0, 0)),
            pl.BlockSpec((BN, 2), lambda i: (i, 0)),
        ],
        out_specs=pl.BlockSpec((BN, D), lambda i: (i, 0)),
        out_shape=jax.ShapeDtypeStruct((Nn, D), jnp.float32),
    )(x, W, degT)


def _tc_next(part, g_prev, degT, b, W, BN, final, b_out=None):
    Nn, D = g_prev.shape

    def body(p_ref, g_ref, deg_ref, b_ref, w_ref, bo_ref, o_ref):
        deg = deg_ref[:, 0:1] + deg_ref[:, 1:2] + 1.0
        dinv = lax.rsqrt(deg)
        agg = p_ref[0] + p_ref[1] + g_ref[...]
        h = jnp.maximum(agg * dinv + b_ref[...], 0.0)
        hw = jnp.dot(h, w_ref[...], preferred_element_type=jnp.float32)
        if final:
            o_ref[...] = hw + bo_ref[...]
        else:
            o_ref[...] = hw * dinv

    if b_out is None:
        b_out = jnp.zeros((1, D), jnp.float32)

    return pl.pallas_call(
        body,
        grid=(Nn // BN,),
        in_specs=[
            pl.BlockSpec((2, BN, D), lambda i: (0, i, 0)),
            pl.BlockSpec((BN, D), lambda i: (i, 0)),
            pl.BlockSpec((BN, 2), lambda i: (i, 0)),
            pl.BlockSpec((1, D), lambda i: (0, 0)),
            pl.BlockSpec((D, D), lambda i: (0, 0)),
            pl.BlockSpec((1, D), lambda i: (0, 0)),
        ],
        out_specs=pl.BlockSpec((BN, D), lambda i: (i, 0)),
        out_shape=jax.ShapeDtypeStruct((Nn, D), jnp.float32),
    )(part, g_prev, degT, b, W, b_out)


@jax.jit
def kernel(x, edge_index, W1, b1, W2, b2, W_lin, b_lin):
    Nn, D = x.shape
    E = edge_index.shape[1]
    NPAD = ((Nn + NW * LANES - 1) // (NW * LANES)) * (NW * LANES)
    BN = 1000

    e_tile = ((E // NW) + GROUP * CHUNK - 1) // (GROUP * CHUNK) * (GROUP * CHUNK)
    E_pad = e_tile * NW
    n_groups = e_tile // (GROUP * CHUNK)
    pad = E_pad - E
    pad_iota = jnp.arange(pad, dtype=jnp.int32)
    src_p = jnp.concatenate([edge_index[0], pad_iota % Nn])
    dst_p = jnp.concatenate([edge_index[1], Nn + pad_iota % (NPAD - Nn)])
    src4 = src_p.reshape(NW, n_groups, GROUP, CHUNK)
    dst4 = dst_p.reshape(NW, n_groups, GROUP, CHUNK)

    deg_parts = _make_deg_kernel(E, NPAD)(edge_index[1])
    degT = deg_parts.T

    g1 = _tc_first(x, W1, degT, BN)
    part1 = _make_agg_kernel(D, E_pad, NPAD)(g1, src4, dst4)

    g2 = _tc_next(part1, g1, degT, b1.reshape(1, D), W2, BN, final=False)
    part2 = _make_agg_kernel(D, E_pad, NPAD)(g2, src4, dst4)

    y = _tc_next(part2, g2, degT, b2.reshape(1, D), W_lin, BN,
                 final=True, b_out=b_lin.reshape(1, D))
    return y


# speedup vs baseline: 3.3313x; 1.1974x over previous
import functools

import jax
import jax.numpy as jnp
from jax import lax
from jax.experimental import pallas as pl
from jax.experimental.pallas import tpu as pltpu
from jax.experimental.pallas import tpu_sc as plsc

NC = 2
NS = 16
NW = NC * NS
LANES = 16

CHUNK = 128
GROUP = 8
RING = 2


def _mesh():
    return plsc.VectorSubcoreMesh(
        core_axis_name="c", subcore_axis_name="s", num_cores=NC, num_subcores=NS
    )


def _make_deg_kernel(E_pad, NPAD):
    e_per = E_pad // NW
    n_chunks = e_per // CHUNK
    n_groups = n_chunks // GROUP
    rows_per_tile = NPAD // NS

    @functools.partial(
        pl.kernel,
        out_type=jax.ShapeDtypeStruct((NC, NPAD), jnp.float32),
        mesh=_mesh(),
        scratch_types=[
            pltpu.VMEM((2, GROUP, CHUNK), jnp.int32),
            pltpu.VMEM((CHUNK,), jnp.float32),
            pltpu.VMEM((rows_per_tile,), jnp.float32),
            pltpu.VMEM_SHARED((NPAD,), jnp.float32),
            pltpu.SemaphoreType.DMA,
            pltpu.SemaphoreType.DMA,
            pltpu.SemaphoreType.DMA,
            pltpu.SemaphoreType.DMA,
        ],
    )
    def deg_kernel(dst_hbm, out_hbm, dst_v, ones_v, zero_v, acc_sh,
                   is0, is1, ss0, ss1):
        isems = (is0, is1)
        ssems = (ss0, ss1)
        cid = lax.axis_index("c")
        sid = lax.axis_index("s")
        wid = sid * NC + cid

        def idx_start(g, p):
            pltpu.async_copy(dst_hbm.at[wid, g], dst_v.at[p], isems[p])

        def idx_wait(g, p):
            pltpu.make_async_copy(dst_hbm.at[wid, g], dst_v.at[p], isems[p]).wait()

        def sc_start(p, k):
            pltpu.async_copy(ones_v, acc_sh.at[dst_v.at[p, k]], ssems[p], add=True)

        def sc_wait(p, k):
            pltpu.make_async_copy(ones_v, acc_sh.at[dst_v.at[p, k]], ssems[p]).wait()

        idx_start(0, 0)
        idx_start(1, 1)

        def fill_ones(i, carry):
            ones_v[pl.ds(i * LANES, LANES)] = jnp.full((LANES,), 1.0, jnp.float32)
            return carry

        lax.fori_loop(0, CHUNK // LANES, fill_ones, 0)

        def fill_zero(i, carry):
            zero_v[pl.ds(i * LANES, LANES)] = jnp.zeros((LANES,), jnp.float32)
            return carry

        lax.fori_loop(0, rows_per_tile // LANES, fill_zero, 0)

        r0 = sid * rows_per_tile
        pltpu.sync_copy(zero_v, acc_sh.at[pl.ds(r0, rows_per_tile)])
        plsc.subcore_barrier()

        for g in range(n_groups):
            p = g % 2
            idx_wait(g, p)
            for k in range(GROUP):
                sc_start(p, k)
            for k in range(GROUP):
                sc_wait(p, k)
            if g + 2 < n_groups:
                idx_start(g + 2, p)

        plsc.subcore_barrier()
        pltpu.sync_copy(
            acc_sh.at[pl.ds(r0, rows_per_tile)],
            out_hbm.at[cid, pl.ds(r0, rows_per_tile)],
        )

    return deg_kernel


def _make_agg_kernel(D, E_pad, NPAD):
    e_per = E_pad // NW
    n_chunks = e_per // CHUNK
    n_groups = n_chunks // GROUP
    rows_per_tile = NPAD // NS

    @functools.partial(
        pl.kernel,
        out_type=jax.ShapeDtypeStruct((NC, NPAD, D), jnp.float32),
        mesh=_mesh(),
        scratch_types=[
            pltpu.VMEM((2, GROUP, CHUNK), jnp.int32),
            pltpu.VMEM((2, GROUP, CHUNK), jnp.int32),
            pltpu.VMEM((RING, CHUNK, D), jnp.float32),
            pltpu.VMEM_SHARED((NPAD, D), jnp.float32),
            pltpu.SemaphoreType.DMA,
            pltpu.SemaphoreType.DMA,
        ]
        + [pltpu.SemaphoreType.DMA] * RING,
    )
    def agg_kernel(g_hbm, src_hbm, dst_hbm, out_hbm,
                   src_v, dst_v, rows_v, acc_sh, is0, is1, *rsems):
        isems = (is0, is1)
        cid = lax.axis_index("c")
        sid = lax.axis_index("s")
        wid = sid * NC + cid

        def idx_start(g, p):
            pltpu.async_copy(src_hbm.at[wid, g], src_v.at[p], isems[p])
            pltpu.async_copy(dst_hbm.at[wid, g], dst_v.at[p], isems[p])

        def idx_wait(g, p):
            pltpu.make_async_copy(src_hbm.at[wid, g], src_v.at[p], isems[p]).wait()
            pltpu.make_async_copy(dst_hbm.at[wid, g], dst_v.at[p], isems[p]).wait()

        def gather_start(p, k, slot):
            pltpu.async_copy(g_hbm.at[src_v.at[p, k]], rows_v.at[slot], rsems[slot])

        def gather_wait(p, k, slot):
            pltpu.make_async_copy(
                g_hbm.at[src_v.at[p, k]], rows_v.at[slot], rsems[slot]
            ).wait()

        def do_group(g, p, start_next_idx, has_next_group):
            for k in range(GROUP):
                slot = k % RING
                gather_wait(p, k, slot)
                pltpu.sync_copy(rows_v.at[slot], acc_sh.at[dst_v.at[p, k]], add=True)
                if k < GROUP - RING:
                    gather_start(p, k + RING, slot)
                else:
                    if has_next_group:
                        if k == GROUP - RING:
                            idx_wait(g + 1, 1 - p)
                        gather_start(1 - p, k - (GROUP - RING), slot)
            if start_next_idx:
                idx_start(g + 2, p)

        idx_start(0, 0)
        idx_start(1, 1)

        zref = rows_v.at[0]

        def fill_zero(i, carry):
            zref[i // (D // LANES), pl.ds((i % (D // LANES)) * LANES, LANES)] = (
                jnp.zeros((LANES,), jnp.float32)
            )
            return carry

        lax.fori_loop(0, CHUNK * (D // LANES), fill_zero, 0)

        r0 = sid * rows_per_tile
        for k in range(rows_per_tile // CHUNK):
            pltpu.sync_copy(zref, acc_sh.at[pl.ds(r0 + k * CHUNK, CHUNK)])

        idx_wait(0, 0)
        for k in range(RING):
            gather_start(0, k, k)
        plsc.subcore_barrier()

        def supergroup(sg, carry):
            do_group(2 * sg, 0, True, True)
            do_group(2 * sg + 1, 1, True, True)
            return carry

        lax.fori_loop(0, (n_groups - 2) // 2, supergroup, 0)
        do_group(n_groups - 2, 0, False, True)
        do_group(n_groups - 1, 1, False, False)

        plsc.subcore_barrier()
        pltpu.sync_copy(
            acc_sh.at[pl.ds(r0, rows_per_tile)],
            out_hbm.at[cid, pl.ds(r0, rows_per_tile)],
        )

    return agg_kernel


def _tc_first(x, W, degT, BN):
    Nn, D = x.shape

    def body(x_ref, w_ref, deg_ref, o_ref):
        deg = deg_ref[:, 0:1] + deg_ref[:, 1:2] + 1.0
        dinv = lax.rsqrt(deg)
        h = jnp.dot(x_ref[...], w_ref[...], preferred_element_type=jnp.float32)
        o_ref[...] = h * dinv

    return pl.pallas_call(
        body,
        grid=(Nn // BN,),
        in_specs=[
            pl.BlockSpec((BN, D), lambda i: (i, 0)),
            pl.BlockSpec((D, D), lambda i: (0, 0)),
            pl.BlockSpec((BN, 2), lambda i: (i, 0)),
        ],
        out_specs=pl.BlockSpec((BN, D), lambda i: (i, 0)),
        out_shape=jax.ShapeDtypeStruct((Nn, D), jnp.float32),
    )(x, W, degT)


def _tc_next(part, g_prev, degT, b, W, BN, final, b_out=None):
    Nn, D = g_prev.shape

    def body(p_ref, g_ref, deg_ref, b_ref, w_ref, bo_ref, o_ref):
        deg = deg_ref[:, 0:1] + deg_ref[:, 1:2] + 1.0
        dinv = lax.rsqrt(deg)
        agg = p_ref[0] + p_ref[1] + g_ref[...]
        h = jnp.maximum(agg * dinv + b_ref[...], 0.0)
        hw = jnp.dot(h, w_ref[...], preferred_element_type=jnp.float32)
        if final:
            o_ref[...] = hw + bo_ref[...]
        else:
            o_ref[...] = hw * dinv

    if b_out is None:
        b_out = jnp.zeros((1, D), jnp.float32)

    return pl.pallas_call(
        body,
        grid=(Nn // BN,),
        in_specs=[
            pl.BlockSpec((2, BN, D), lambda i: (0, i, 0)),
            pl.BlockSpec((BN, D), lambda i: (i, 0)),
            pl.BlockSpec((BN, 2), lambda i: (i, 0)),
            pl.BlockSpec((1, D), lambda i: (0, 0)),
            pl.BlockSpec((D, D), lambda i: (0, 0)),
            pl.BlockSpec((1, D), lambda i: (0, 0)),
        ],
        out_specs=pl.BlockSpec((BN, D), lambda i: (i, 0)),
        out_shape=jax.ShapeDtypeStruct((Nn, D), jnp.float32),
    )(part, g_prev, degT, b, W, b_out)


@jax.jit
def kernel(x, edge_index, W1, b1, W2, b2, W_lin, b_lin):
    Nn, D = x.shape
    E = edge_index.shape[1]
    NPAD = ((Nn + NW * LANES - 1) // (NW * LANES)) * (NW * LANES)
    BN = 1000

    e_tile = ((E // NW) + GROUP * CHUNK - 1) // (GROUP * CHUNK) * (GROUP * CHUNK)
    E_pad = e_tile * NW
    n_groups = e_tile // (GROUP * CHUNK)
    pad = E_pad - E
    pad_iota = jnp.arange(pad, dtype=jnp.int32)
    src_p = jnp.concatenate([edge_index[0], pad_iota % Nn])
    dst_p = jnp.concatenate([edge_index[1], Nn + pad_iota % (NPAD - Nn)])
    src4 = src_p.reshape(NW, n_groups, GROUP, CHUNK)
    dst4 = dst_p.reshape(NW, n_groups, GROUP, CHUNK)

    deg_parts = _make_deg_kernel(E_pad, NPAD)(dst4)
    degT = deg_parts.T

    g1 = _tc_first(x, W1, degT, BN)
    part1 = _make_agg_kernel(D, E_pad, NPAD)(g1, src4, dst4)

    g2 = _tc_next(part1, g1, degT, b1.reshape(1, D), W2, BN, final=False)
    part2 = _make_agg_kernel(D, E_pad, NPAD)(g2, src4, dst4)

    y = _tc_next(part2, g2, degT, b2.reshape(1, D), W_lin, BN,
                 final=True, b_out=b_lin.reshape(1, D))
    return y


# speedup vs baseline: 3.3354x; 1.0012x over previous
import functools

import jax
import jax.numpy as jnp
from jax import lax
from jax.experimental import pallas as pl
from jax.experimental.pallas import tpu as pltpu
from jax.experimental.pallas import tpu_sc as plsc

NC = 2
NS = 16
NW = NC * NS
LANES = 16

CHUNK = 128
GROUP = 8
RING = 2


def _mesh():
    return plsc.VectorSubcoreMesh(
        core_axis_name="c", subcore_axis_name="s", num_cores=NC, num_subcores=NS
    )


def _make_deg_kernel(E_pad, NPAD):
    e_per = E_pad // NW
    n_chunks = e_per // CHUNK
    n_groups = n_chunks // GROUP
    rows_per_tile = NPAD // NS

    @functools.partial(
        pl.kernel,
        out_type=jax.ShapeDtypeStruct((NC, NPAD), jnp.float32),
        mesh=_mesh(),
        scratch_types=[
            pltpu.VMEM((2, GROUP, CHUNK), jnp.int32),
            pltpu.VMEM((CHUNK,), jnp.float32),
            pltpu.VMEM((rows_per_tile,), jnp.float32),
            pltpu.VMEM_SHARED((NPAD,), jnp.float32),
            pltpu.SemaphoreType.DMA,
            pltpu.SemaphoreType.DMA,
            pltpu.SemaphoreType.DMA,
            pltpu.SemaphoreType.DMA,
        ],
    )
    def deg_kernel(dst_hbm, out_hbm, dst_v, ones_v, zero_v, acc_sh,
                   is0, is1, ss0, ss1):
        isems = (is0, is1)
        ssems = (ss0, ss1)
        cid = lax.axis_index("c")
        sid = lax.axis_index("s")
        wid = sid * NC + cid

        def idx_start(g, p):
            pltpu.async_copy(dst_hbm.at[wid, g], dst_v.at[p], isems[p])

        def idx_wait(g, p):
            pltpu.make_async_copy(dst_hbm.at[wid, g], dst_v.at[p], isems[p]).wait()

        def sc_start(p, k):
            pltpu.async_copy(ones_v, acc_sh.at[dst_v.at[p, k]], ssems[p], add=True)

        def sc_wait(p, k):
            pltpu.make_async_copy(ones_v, acc_sh.at[dst_v.at[p, k]], ssems[p]).wait()

        idx_start(0, 0)
        idx_start(1, 1)

        def fill_ones(i, carry):
            ones_v[pl.ds(i * LANES, LANES)] = jnp.full((LANES,), 1.0, jnp.float32)
            return carry

        lax.fori_loop(0, CHUNK // LANES, fill_ones, 0)

        def fill_zero(i, carry):
            zero_v[pl.ds(i * LANES, LANES)] = jnp.zeros((LANES,), jnp.float32)
            return carry

        lax.fori_loop(0, rows_per_tile // LANES, fill_zero, 0)

        r0 = sid * rows_per_tile
        pltpu.sync_copy(zero_v, acc_sh.at[pl.ds(r0, rows_per_tile)])
        plsc.subcore_barrier()

        for g in range(n_groups):
            p = g % 2
            idx_wait(g, p)
            for k in range(GROUP):
                sc_start(p, k)
            for k in range(GROUP):
                sc_wait(p, k)
            if g + 2 < n_groups:
                idx_start(g + 2, p)

        plsc.subcore_barrier()
        pltpu.sync_copy(
            acc_sh.at[pl.ds(r0, rows_per_tile)],
            out_hbm.at[cid, pl.ds(r0, rows_per_tile)],
        )

    return deg_kernel


def _make_agg_kernel(D, E_pad, NPAD):
    e_per = E_pad // NW
    n_chunks = e_per // CHUNK
    n_groups = n_chunks // GROUP
    rows_per_tile = NPAD // NS

    @functools.partial(
        pl.kernel,
        out_type=jax.ShapeDtypeStruct((NC, NPAD, D), jnp.float32),
        mesh=_mesh(),
        scratch_types=[
            pltpu.VMEM((2, GROUP, CHUNK), jnp.int32),
            pltpu.VMEM((2, GROUP, CHUNK), jnp.int32),
            pltpu.VMEM((RING, CHUNK, D), jnp.float32),
            pltpu.VMEM_SHARED((NPAD, D), jnp.float32),
            pltpu.SemaphoreType.DMA,
            pltpu.SemaphoreType.DMA,
        ]
        + [pltpu.SemaphoreType.DMA] * RING,
    )
    def agg_kernel(g_hbm, src_hbm, dst_hbm, out_hbm,
                   src_v, dst_v, rows_v, acc_sh, is0, is1, *rsems):
        isems = (is0, is1)
        cid = lax.axis_index("c")
        sid = lax.axis_index("s")
        wid = sid * NC + cid

        def idx_start(g, p):
            pltpu.async_copy(src_hbm.at[wid, g], src_v.at[p], isems[p])
            pltpu.async_copy(dst_hbm.at[wid, g], dst_v.at[p], isems[p])

        def idx_wait(g, p):
            pltpu.make_async_copy(src_hbm.at[wid, g], src_v.at[p], isems[p]).wait()
            pltpu.make_async_copy(dst_hbm.at[wid, g], dst_v.at[p], isems[p]).wait()

        def gather_start(p, k, slot):
            pltpu.async_copy(g_hbm.at[src_v.at[p, k]], rows_v.at[slot], rsems[slot])

        def gather_wait(p, k, slot):
            pltpu.make_async_copy(
                g_hbm.at[src_v.at[p, k]], rows_v.at[slot], rsems[slot]
            ).wait()

        def do_group(g, p, start_next_idx, has_next_group):
            for k in range(GROUP):
                slot = k % RING
                gather_wait(p, k, slot)
                pltpu.sync_copy(rows_v.at[slot], acc_sh.at[dst_v.at[p, k]], add=True)
                if k < GROUP - RING:
                    gather_start(p, k + RING, slot)
                else:
                    if has_next_group:
                        if k == GROUP - RING:
                            idx_wait(g + 1, 1 - p)
                        gather_start(1 - p, k - (GROUP - RING), slot)
            if start_next_idx:
                idx_start(g + 2, p)

        idx_start(0, 0)
        idx_start(1, 1)

        zref = rows_v.at[0]

        def fill_zero(i, carry):
            zref[i // (D // LANES), pl.ds((i % (D // LANES)) * LANES, LANES)] = (
                jnp.zeros((LANES,), jnp.float32)
            )
            return carry

        lax.fori_loop(0, CHUNK * (D // LANES), fill_zero, 0)

        r0 = sid * rows_per_tile
        for k in range(rows_per_tile // CHUNK):
            pltpu.sync_copy(zref, acc_sh.at[pl.ds(r0 + k * CHUNK, CHUNK)])

        idx_wait(0, 0)
        for k in range(RING):
            gather_start(0, k, k)
        plsc.subcore_barrier()

        def supergroup(sg, carry):
            do_group(2 * sg, 0, True, True)
            do_group(2 * sg + 1, 1, True, True)
            return carry

        lax.fori_loop(0, (n_groups - 2) // 2, supergroup, 0)
        do_group(n_groups - 2, 0, False, True)
        do_group(n_groups - 1, 1, False, False)

        plsc.subcore_barrier()
        pltpu.sync_copy(
            acc_sh.at[pl.ds(r0, rows_per_tile)],
            out_hbm.at[cid, pl.ds(r0, rows_per_tile)],
        )

    return agg_kernel


def _tc_first(x, W, degT, BN):
    Nn, D = x.shape

    def body(x_ref, w_ref, deg_ref, o_ref):
        deg = deg_ref[:, 0:1] + deg_ref[:, 1:2] + 1.0
        dinv = lax.rsqrt(deg)
        h = jnp.dot(x_ref[...].astype(jnp.bfloat16), w_ref[...].astype(jnp.bfloat16),
                    preferred_element_type=jnp.float32)
        o_ref[...] = h * dinv

    return pl.pallas_call(
        body,
        grid=(Nn // BN,),
        in_specs=[
            pl.BlockSpec((BN, D), lambda i: (i, 0)),
            pl.BlockSpec((D, D), lambda i: (0, 0)),
            pl.BlockSpec((BN, 2), lambda i: (i, 0)),
        ],
        out_specs=pl.BlockSpec((BN, D), lambda i: (i, 0)),
        out_shape=jax.ShapeDtypeStruct((Nn, D), jnp.float32),
    )(x, W, degT)


def _tc_next(part, g_prev, degT, b, W, BN, final, b_out=None):
    Nn, D = g_prev.shape

    def body(p_ref, g_ref, deg_ref, b_ref, w_ref, bo_ref, o_ref):
        deg = deg_ref[:, 0:1] + deg_ref[:, 1:2] + 1.0
        dinv = lax.rsqrt(deg)
        agg = p_ref[0] + p_ref[1] + g_ref[...]
        h = jnp.maximum(agg * dinv + b_ref[...], 0.0)
        hw = jnp.dot(h.astype(jnp.bfloat16), w_ref[...].astype(jnp.bfloat16),
                     preferred_element_type=jnp.float32)
        if final:
            o_ref[...] = hw + bo_ref[...]
        else:
            o_ref[...] = hw * dinv

    if b_out is None:
        b_out = jnp.zeros((1, D), jnp.float32)

    return pl.pallas_call(
        body,
        grid=(Nn // BN,),
        in_specs=[
            pl.BlockSpec((2, BN, D), lambda i: (0, i, 0)),
            pl.BlockSpec((BN, D), lambda i: (i, 0)),
            pl.BlockSpec((BN, 2), lambda i: (i, 0)),
            pl.BlockSpec((1, D), lambda i: (0, 0)),
            pl.BlockSpec((D, D), lambda i: (0, 0)),
            pl.BlockSpec((1, D), lambda i: (0, 0)),
        ],
        out_specs=pl.BlockSpec((BN, D), lambda i: (i, 0)),
        out_shape=jax.ShapeDtypeStruct((Nn, D), jnp.float32),
    )(part, g_prev, degT, b, W, b_out)


@jax.jit
def kernel(x, edge_index, W1, b1, W2, b2, W_lin, b_lin):
    Nn, D = x.shape
    E = edge_index.shape[1]
    NPAD = ((Nn + NW * LANES - 1) // (NW * LANES)) * (NW * LANES)
    BN = 1000

    e_tile = ((E // NW) + GROUP * CHUNK - 1) // (GROUP * CHUNK) * (GROUP * CHUNK)
    E_pad = e_tile * NW
    n_groups = e_tile // (GROUP * CHUNK)
    pad = E_pad - E
    pad_iota = jnp.arange(pad, dtype=jnp.int32)
    src_p = jnp.concatenate([edge_index[0], pad_iota % Nn])
    dst_p = jnp.concatenate([edge_index[1], Nn + pad_iota % (NPAD - Nn)])
    src4 = src_p.reshape(NW, n_groups, GROUP, CHUNK)
    dst4 = dst_p.reshape(NW, n_groups, GROUP, CHUNK)

    deg_parts = _make_deg_kernel(E_pad, NPAD)(dst4)
    degT = deg_parts.T

    g1 = _tc_first(x, W1, degT, BN)
    part1 = _make_agg_kernel(D, E_pad, NPAD)(g1, src4, dst4)

    g2 = _tc_next(part1, g1, degT, b1.reshape(1, D), W2, BN, final=False)
    part2 = _make_agg_kernel(D, E_pad, NPAD)(g2, src4, dst4)

    y = _tc_next(part2, g2, degT, b2.reshape(1, D), W_lin, BN,
                 final=True, b_out=b_lin.reshape(1, D))
    return y


# speedup vs baseline: 3.3389x; 1.0011x over previous
import functools

import jax
import jax.numpy as jnp
from jax import lax
from jax.experimental import pallas as pl
from jax.experimental.pallas import tpu as pltpu
from jax.experimental.pallas import tpu_sc as plsc

NC = 2
NS = 16
NW = NC * NS
LANES = 16

CHUNK = 128
GROUP = 8
RING = 2


def _mesh():
    return plsc.VectorSubcoreMesh(
        core_axis_name="c", subcore_axis_name="s", num_cores=NC, num_subcores=NS
    )


def _make_deg_kernel(E_pad, NPAD):
    e_per = E_pad // NW
    n_chunks = e_per // CHUNK
    n_groups = n_chunks // GROUP
    rows_per_tile = NPAD // NS

    @functools.partial(
        pl.kernel,
        out_type=jax.ShapeDtypeStruct((NC, NPAD), jnp.float32),
        mesh=_mesh(),
        scratch_types=[
            pltpu.VMEM((2, GROUP, CHUNK), jnp.int32),
            pltpu.VMEM((CHUNK,), jnp.float32),
            pltpu.VMEM((rows_per_tile,), jnp.float32),
            pltpu.VMEM_SHARED((NPAD,), jnp.float32),
            pltpu.SemaphoreType.DMA,
            pltpu.SemaphoreType.DMA,
            pltpu.SemaphoreType.DMA,
            pltpu.SemaphoreType.DMA,
        ],
    )
    def deg_kernel(dst_hbm, out_hbm, dst_v, ones_v, zero_v, acc_sh,
                   is0, is1, ss0, ss1):
        isems = (is0, is1)
        ssems = (ss0, ss1)
        cid = lax.axis_index("c")
        sid = lax.axis_index("s")
        wid = sid * NC + cid

        def idx_start(g, p):
            pltpu.async_copy(dst_hbm.at[wid, g], dst_v.at[p], isems[p])

        def idx_wait(g, p):
            pltpu.make_async_copy(dst_hbm.at[wid, g], dst_v.at[p], isems[p]).wait()

        def sc_start(p, k):
            pltpu.async_copy(ones_v, acc_sh.at[dst_v.at[p, k]], ssems[p], add=True)

        def sc_wait(p, k):
            pltpu.make_async_copy(ones_v, acc_sh.at[dst_v.at[p, k]], ssems[p]).wait()

        idx_start(0, 0)
        idx_start(1, 1)

        def fill_ones(i, carry):
            ones_v[pl.ds(i * LANES, LANES)] = jnp.full((LANES,), 1.0, jnp.float32)
            return carry

        lax.fori_loop(0, CHUNK // LANES, fill_ones, 0)

        def fill_zero(i, carry):
            zero_v[pl.ds(i * LANES, LANES)] = jnp.zeros((LANES,), jnp.float32)
            return carry

        lax.fori_loop(0, rows_per_tile // LANES, fill_zero, 0)

        r0 = sid * rows_per_tile
        pltpu.sync_copy(zero_v, acc_sh.at[pl.ds(r0, rows_per_tile)])
        plsc.subcore_barrier()

        for g in range(n_groups):
            p = g % 2
            idx_wait(g, p)
            for k in range(GROUP):
                sc_start(p, k)
            for k in range(GROUP):
                sc_wait(p, k)
            if g + 2 < n_groups:
                idx_start(g + 2, p)

        plsc.subcore_barrier()
        pltpu.sync_copy(
            acc_sh.at[pl.ds(r0, rows_per_tile)],
            out_hbm.at[cid, pl.ds(r0, rows_per_tile)],
        )

    return deg_kernel


def _make_agg_kernel(D, E_pad, NPAD):
    e_per = E_pad // NW
    n_chunks = e_per // CHUNK
    n_groups = n_chunks // GROUP
    rows_per_tile = NPAD // NS

    @functools.partial(
        pl.kernel,
        out_type=jax.ShapeDtypeStruct((NC, NPAD, D), jnp.float32),
        mesh=_mesh(),
        scratch_types=[
            pltpu.VMEM((2, GROUP, CHUNK), jnp.int32),
            pltpu.VMEM((2, GROUP, CHUNK), jnp.int32),
            pltpu.VMEM((RING, CHUNK, D), jnp.float32),
            pltpu.VMEM_SHARED((NPAD, D), jnp.float32),
            pltpu.SemaphoreType.DMA,
            pltpu.SemaphoreType.DMA,
        ]
        + [pltpu.SemaphoreType.DMA] * RING,
    )
    def agg_kernel(g_hbm, src_hbm, dst_hbm, out_hbm,
                   src_v, dst_v, rows_v, acc_sh, is0, is1, *rsems):
        isems = (is0, is1)
        cid = lax.axis_index("c")
        sid = lax.axis_index("s")
        wid = sid * NC + cid

        def idx_start(g, p):
            pltpu.async_copy(src_hbm.at[wid, g], src_v.at[p], isems[p])
            pltpu.async_copy(dst_hbm.at[wid, g], dst_v.at[p], isems[p])

        def idx_wait(g, p):
            pltpu.make_async_copy(src_hbm.at[wid, g], src_v.at[p], isems[p]).wait()
            pltpu.make_async_copy(dst_hbm.at[wid, g], dst_v.at[p], isems[p]).wait()

        def gather_start(p, k, slot):
            pltpu.async_copy(g_hbm.at[src_v.at[p, k]], rows_v.at[slot], rsems[slot])

        def gather_wait(p, k, slot):
            pltpu.make_async_copy(
                g_hbm.at[src_v.at[p, k]], rows_v.at[slot], rsems[slot]
            ).wait()

        def do_group(g, p, start_next_idx, has_next_group):
            for k in range(GROUP):
                slot = k % RING
                gather_wait(p, k, slot)
                pltpu.sync_copy(rows_v.at[slot], acc_sh.at[dst_v.at[p, k]], add=True)
                if k < GROUP - RING:
                    gather_start(p, k + RING, slot)
                else:
                    if has_next_group:
                        if k == GROUP - RING:
                            idx_wait(g + 1, 1 - p)
                        gather_start(1 - p, k - (GROUP - RING), slot)
            if start_next_idx:
                idx_start(g + 2, p)

        idx_start(0, 0)
        idx_start(1, 1)

        zref = rows_v.at[0]

        def fill_zero(i, carry):
            zref[i // (D // LANES), pl.ds((i % (D // LANES)) * LANES, LANES)] = (
                jnp.zeros((LANES,), jnp.float32)
            )
            return carry

        lax.fori_loop(0, CHUNK * (D // LANES), fill_zero, 0)

        r0 = sid * rows_per_tile
        for k in range(rows_per_tile // CHUNK):
            pltpu.sync_copy(zref, acc_sh.at[pl.ds(r0 + k * CHUNK, CHUNK)])

        idx_wait(0, 0)
        for k in range(RING):
            gather_start(0, k, k)
        plsc.subcore_barrier()

        def supergroup(sg, carry):
            do_group(2 * sg, 0, True, True)
            do_group(2 * sg + 1, 1, True, True)
            return carry

        lax.fori_loop(0, (n_groups - 2) // 2, supergroup, 0)
        do_group(n_groups - 2, 0, False, True)
        do_group(n_groups - 1, 1, False, False)

        plsc.subcore_barrier()
        pltpu.sync_copy(
            acc_sh.at[pl.ds(r0, rows_per_tile)],
            out_hbm.at[cid, pl.ds(r0, rows_per_tile)],
        )

    return agg_kernel


def _tc_first(x, W, degT, BN):
    Nn, D = x.shape

    def body(x_ref, w_ref, deg_ref, o_ref):
        deg = deg_ref[:, 0:1] + deg_ref[:, 1:2] + 1.0
        dinv = lax.rsqrt(deg)
        h = jnp.dot(x_ref[...], w_ref[...], preferred_element_type=jnp.float32)
        o_ref[...] = h * dinv

    return pl.pallas_call(
        body,
        grid=(Nn // BN,),
        in_specs=[
            pl.BlockSpec((BN, D), lambda i: (i, 0)),
            pl.BlockSpec((D, D), lambda i: (0, 0)),
            pl.BlockSpec((BN, 2), lambda i: (i, 0)),
        ],
        out_specs=pl.BlockSpec((BN, D), lambda i: (i, 0)),
        out_shape=jax.ShapeDtypeStruct((Nn, D), jnp.float32),
    )(x, W, degT)


def _tc_next(part, g_prev, degT, b, W, BN, final, b_out=None):
    Nn, D = g_prev.shape

    def body(p_ref, g_ref, deg_ref, b_ref, w_ref, bo_ref, o_ref):
        deg = deg_ref[:, 0:1] + deg_ref[:, 1:2] + 1.0
        dinv = lax.rsqrt(deg)
        agg = p_ref[0] + p_ref[1] + g_ref[...]
        h = jnp.maximum(agg * dinv + b_ref[...], 0.0)
        hw = jnp.dot(h, w_ref[...], preferred_element_type=jnp.float32)
        if final:
            o_ref[...] = hw + bo_ref[...]
        else:
            o_ref[...] = hw * dinv

    if b_out is None:
        b_out = jnp.zeros((1, D), jnp.float32)

    return pl.pallas_call(
        body,
        grid=(Nn // BN,),
        in_specs=[
            pl.BlockSpec((2, BN, D), lambda i: (0, i, 0)),
            pl.BlockSpec((BN, D), lambda i: (i, 0)),
            pl.BlockSpec((BN, 2), lambda i: (i, 0)),
            pl.BlockSpec((1, D), lambda i: (0, 0)),
            pl.BlockSpec((D, D), lambda i: (0, 0)),
            pl.BlockSpec((1, D), lambda i: (0, 0)),
        ],
        out_specs=pl.BlockSpec((BN, D), lambda i: (i, 0)),
        out_shape=jax.ShapeDtypeStruct((Nn, D), jnp.float32),
    )(part, g_prev, degT, b, W, b_out)


@jax.jit
def kernel(x, edge_index, W1, b1, W2, b2, W_lin, b_lin):
    Nn, D = x.shape
    E = edge_index.shape[1]
    NPAD = ((Nn + NW * LANES - 1) // (NW * LANES)) * (NW * LANES)
    BN = 1000

    e_tile = ((E // NW) + GROUP * CHUNK - 1) // (GROUP * CHUNK) * (GROUP * CHUNK)
    E_pad = e_tile * NW
    n_groups = e_tile // (GROUP * CHUNK)
    pad = E_pad - E
    pad_iota = jnp.arange(pad, dtype=jnp.int32)
    src_p = jnp.concatenate([edge_index[0], pad_iota % Nn])
    dst_p = jnp.concatenate([edge_index[1], Nn + pad_iota % (NPAD - Nn)])
    src4 = src_p.reshape(NW, n_groups, GROUP, CHUNK)
    dst4 = dst_p.reshape(NW, n_groups, GROUP, CHUNK)

    deg_parts = _make_deg_kernel(E_pad, NPAD)(dst4)
    degT = deg_parts.T

    g1 = _tc_first(x, W1, degT, BN)
    part1 = _make_agg_kernel(D, E_pad, NPAD)(g1, src4, dst4)

    g2 = _tc_next(part1, g1, degT, b1.reshape(1, D), W2, BN, final=False)
    part2 = _make_agg_kernel(D, E_pad, NPAD)(g2, src4, dst4)

    y = _tc_next(part2, g2, degT, b2.reshape(1, D), W_lin, BN,
                 final=True, b_out=b_lin.reshape(1, D))
    return y
